# Initial kernel scaffold; baseline (speedup 1.0000x reference)
import jax
import jax.numpy as jnp
from jax.experimental import pallas as pl


def kernel(xyz, feature):
    raise NotImplementedError("write your pallas kernel here")


# speedup vs baseline: 1.1272x; 1.1272x over previous
import functools

import jax
import jax.numpy as jnp
from jax import lax
from jax.experimental import pallas as pl
from jax.experimental.pallas import tpu as pltpu
from jax.experimental.pallas import tpu_sc as plsc

B = 16
N = 16384
C = 256
NSAMPLE = 4096

_NC = 2
_NS = 16
_NW = _NC * _NS
_RPW = B * NSAMPLE // _NW
_CHUNK = 128
_NCHUNK = _RPW // _CHUNK
_HALF = _RPW


def _gather_kernel(feat_hbm, xyz_hbm, idxf_hbm, idx3_hbm,
                   outf_hbm, outx_hbm,
                   idxf_v, idx3_v, xyz_v, outx_v, buf0, buf1,
                   sem0, sem1):
    wid = lax.axis_index("s") * _NC + lax.axis_index("c")
    b = wid // 2

    pltpu.sync_copy(idxf_hbm.at[wid], idxf_v)
    pltpu.sync_copy(idx3_hbm.at[wid], idx3_v)

    bufs = (buf0, buf1)
    sems = (sem0, sem1)
    cps = [None, None]
    cps[0] = pltpu.async_copy(feat_hbm.at[idxf_v.at[0]], buf0, sem0)

    pltpu.sync_copy(xyz_hbm.at[b], xyz_v)

    def _xyz_body(j, carry):
        ids = idx3_v[pl.ds(j * 16, 16)]
        for d in range(3):
            outx_v[d, pl.ds(j * 16, 16)] = plsc.load_gather(xyz_v, [ids + d])
        return carry

    lax.fori_loop(0, _RPW // 16, _xyz_body, 0)
    pltpu.sync_copy(outx_v, outx_hbm.at[wid])

    row_base = wid * _RPW
    for i in range(_NCHUNK):
        cps[i % 2].wait()
        nxt = i + 1
        if nxt < _NCHUNK:
            cps[nxt % 2] = pltpu.async_copy(
                feat_hbm.at[idxf_v.at[nxt]], bufs[nxt % 2], sems[nxt % 2])
        pltpu.sync_copy(bufs[i % 2],
                        outf_hbm.at[pl.ds(row_base + i * _CHUNK, _CHUNK)])


@functools.partial(
    pl.kernel,
    out_type=(
        jax.ShapeDtypeStruct((B * NSAMPLE, C), jnp.float32),
        jax.ShapeDtypeStruct((_NW, 3, _RPW), jnp.float32),
    ),
    mesh=plsc.VectorSubcoreMesh(core_axis_name="c", subcore_axis_name="s"),
    compiler_params=pltpu.CompilerParams(needs_layout_passes=False),
    scratch_types=[
        pltpu.VMEM((_NCHUNK, _CHUNK), jnp.int32),
        pltpu.VMEM((_RPW,), jnp.int32),
        pltpu.VMEM((N * 3,), jnp.float32),
        pltpu.VMEM((3, _RPW), jnp.float32),
        pltpu.VMEM((_CHUNK, C), jnp.float32),
        pltpu.VMEM((_CHUNK, C), jnp.float32),
        pltpu.SemaphoreType.DMA,
        pltpu.SemaphoreType.DMA,
    ],
)
def _sc_gather(feat_hbm, xyz_hbm, idxf_hbm, idx3_hbm, outf_hbm, outx_hbm,
               idxf_v, idx3_v, xyz_v, outx_v, buf0, buf1, sem0, sem1):
    _gather_kernel(feat_hbm, xyz_hbm, idxf_hbm, idx3_hbm,
                   outf_hbm, outx_hbm,
                   idxf_v, idx3_v, xyz_v, outx_v, buf0, buf1, sem0, sem1)


def kernel(xyz, feature):
    keys = jax.random.split(jax.random.key(42), B)
    perm = jax.vmap(lambda k: jax.random.permutation(k, N))(keys)
    idx = perm[:, :NSAMPLE].astype(jnp.int32)

    idx_flat = idx + (jnp.arange(B, dtype=jnp.int32) * N)[:, None]
    idxf = idx_flat.reshape(_NW, _NCHUNK, _CHUNK)
    idx3 = (idx * 3).reshape(_NW, _RPW)

    featf = feature.reshape(B * N, C)
    xyzf = xyz.reshape(B, N * 3)

    outf, outx = _sc_gather(featf, xyzf, idxf, idx3)

    feat_s = outf.reshape(B, NSAMPLE, C)
    xyz_s = (outx.reshape(B, 2, 3, _RPW)
                 .transpose(0, 1, 3, 2)
                 .reshape(B, NSAMPLE, 3))
    return (xyz_s, feat_s)


# speedup vs baseline: 3.0951x; 2.7458x over previous
import functools

import jax
import jax.numpy as jnp
import numpy as np
from jax import lax
from jax.experimental import pallas as pl
from jax.experimental.pallas import tpu as pltpu
from jax.experimental.pallas import tpu_sc as plsc

B = 16
N = 16384
C = 256
NSAMPLE = 4096

_NC = 2
_NS = 16
_NW = _NC * _NS
_RPW = B * NSAMPLE // _NW
_CHUNK = 128
_NCHUNK = _RPW // _CHUNK
_HALF = _RPW


def _gather_kernel(feat_hbm, xyz_hbm, idxf_hbm, idx3_hbm,
                   outf_hbm, outx_hbm,
                   idxf_v, idx3_v, xyz_v, outx_v, buf0, buf1,
                   sem0, sem1):
    wid = lax.axis_index("s") * _NC + lax.axis_index("c")
    b = wid // 2

    pltpu.sync_copy(idxf_hbm.at[wid], idxf_v)
    pltpu.sync_copy(idx3_hbm.at[wid], idx3_v)

    bufs = (buf0, buf1)
    sems = (sem0, sem1)
    cps = [None, None]
    cps[0] = pltpu.async_copy(feat_hbm.at[idxf_v.at[0]], buf0, sem0)

    pltpu.sync_copy(xyz_hbm.at[b], xyz_v)

    def _xyz_body(j, carry):
        ids = idx3_v[pl.ds(j * 16, 16)]
        for d in range(3):
            outx_v[d, pl.ds(j * 16, 16)] = plsc.load_gather(xyz_v, [ids + d])
        return carry

    lax.fori_loop(0, _RPW // 16, _xyz_body, 0)
    pltpu.sync_copy(outx_v, outx_hbm.at[wid])

    row_base = wid * _RPW
    for i in range(_NCHUNK):
        cps[i % 2].wait()
        nxt = i + 1
        if nxt < _NCHUNK:
            cps[nxt % 2] = pltpu.async_copy(
                feat_hbm.at[idxf_v.at[nxt]], bufs[nxt % 2], sems[nxt % 2])
        pltpu.sync_copy(bufs[i % 2],
                        outf_hbm.at[pl.ds(row_base + i * _CHUNK, _CHUNK)])


@functools.partial(
    pl.kernel,
    out_type=(
        jax.ShapeDtypeStruct((B * NSAMPLE, C), jnp.float32),
        jax.ShapeDtypeStruct((_NW, 3, _RPW), jnp.float32),
    ),
    mesh=plsc.VectorSubcoreMesh(core_axis_name="c", subcore_axis_name="s"),
    compiler_params=pltpu.CompilerParams(needs_layout_passes=False),
    scratch_types=[
        pltpu.VMEM((_NCHUNK, _CHUNK), jnp.int32),
        pltpu.VMEM((_RPW,), jnp.int32),
        pltpu.VMEM((N * 3,), jnp.float32),
        pltpu.VMEM((3, _RPW), jnp.float32),
        pltpu.VMEM((_CHUNK, C), jnp.float32),
        pltpu.VMEM((_CHUNK, C), jnp.float32),
        pltpu.SemaphoreType.DMA,
        pltpu.SemaphoreType.DMA,
    ],
)
def _sc_gather(feat_hbm, xyz_hbm, idxf_hbm, idx3_hbm, outf_hbm, outx_hbm,
               idxf_v, idx3_v, xyz_v, outx_v, buf0, buf1, sem0, sem1):
    _gather_kernel(feat_hbm, xyz_hbm, idxf_hbm, idx3_hbm,
                   outf_hbm, outx_hbm,
                   idxf_v, idx3_v, xyz_v, outx_v, buf0, buf1, sem0, sem1)


_IDX_CACHE = None


def _sample_idx() -> np.ndarray:
    global _IDX_CACHE
    if _IDX_CACHE is None:
        with jax.ensure_compile_time_eval():
            keys = jax.random.split(jax.random.key(42), B)
            perm = jax.vmap(lambda k: jax.random.permutation(k, N))(keys)
            _IDX_CACHE = np.asarray(perm[:, :NSAMPLE]).astype(np.int32)
    return _IDX_CACHE


def kernel(xyz, feature):
    idx = _sample_idx()

    idx_flat = idx + (np.arange(B, dtype=np.int32) * N)[:, None]
    idxf = jnp.asarray(idx_flat.reshape(_NW, _NCHUNK, _CHUNK))
    idx3 = jnp.asarray((idx * 3).reshape(_NW, _RPW))

    featf = feature.reshape(B * N, C)
    xyzf = xyz.reshape(B, N * 3)

    outf, outx = _sc_gather(featf, xyzf, idxf, idx3)

    feat_s = outf.reshape(B, NSAMPLE, C)
    xyz_s = (outx.reshape(B, 2, 3, _RPW)
                 .transpose(0, 1, 3, 2)
                 .reshape(B, NSAMPLE, 3))
    return (xyz_s, feat_s)


# speedup vs baseline: 5.2021x; 1.6808x over previous
import functools

import jax
import jax.numpy as jnp
import numpy as np
from jax import lax
from jax.experimental import pallas as pl
from jax.experimental.pallas import tpu as pltpu
from jax.experimental.pallas import tpu_sc as plsc

B = 16
N = 16384
C = 256
NSAMPLE = 4096

_NC = 2
_NS = 16
_NW = _NC * _NS
_RPW = B * NSAMPLE // _NW
_CHUNK = 64
_NCHUNK = _RPW // _CHUNK


def _gather_body(feat_hbm, xyz_hbm, idxf_hbm, idxl_hbm,
                 outf_hbm, outx_hbm,
                 idxf_v, idxl_v, xyz_v, outx_v, buf0, buf1,
                 sem0, sem1):
    wid = lax.axis_index("s") * _NC + lax.axis_index("c")
    b = wid // 2

    pltpu.sync_copy(idxf_hbm.at[wid], idxf_v)
    pltpu.sync_copy(idxl_hbm.at[wid], idxl_v)

    bufs = (buf0, buf1)
    sems = (sem0, sem1)
    cps = [None, None]
    cps[0] = pltpu.async_copy(feat_hbm.at[idxf_v.at[0]], buf0, sem0)

    for d in range(3):
        pltpu.sync_copy(xyz_hbm.at[d, pl.ds(b, 1)],
                        xyz_v.at[pl.ds(d, 1)])

    def _xyz_body(j, carry):
        ids = idxl_v[pl.ds(j * 16, 16)]
        for d in range(3):
            dvec = jnp.full((16,), d, jnp.int32)
            outx_v[d, pl.ds(j * 16, 16)] = plsc.load_gather(
                xyz_v, [dvec, ids])
        return carry

    lax.fori_loop(0, _RPW // 16, _xyz_body, 0)
    pltpu.sync_copy(outx_v, outx_hbm.at[wid])

    row_base = wid * _RPW
    for i in range(_NCHUNK):
        cps[i % 2].wait()
        nxt = i + 1
        if nxt < _NCHUNK:
            cps[nxt % 2] = pltpu.async_copy(
                feat_hbm.at[idxf_v.at[nxt]], bufs[nxt % 2], sems[nxt % 2])
        pltpu.sync_copy(bufs[i % 2],
                        outf_hbm.at[pl.ds(row_base + i * _CHUNK, _CHUNK)])


@functools.partial(
    pl.kernel,
    out_type=(
        jax.ShapeDtypeStruct((B * NSAMPLE, C), jnp.float32),
        jax.ShapeDtypeStruct((_NW, 3, _RPW), jnp.float32),
    ),
    mesh=plsc.VectorSubcoreMesh(core_axis_name="c", subcore_axis_name="s"),
    compiler_params=pltpu.CompilerParams(needs_layout_passes=False),
    scratch_types=[
        pltpu.VMEM((_NCHUNK, _CHUNK), jnp.int32),
        pltpu.VMEM((_RPW,), jnp.int32),
        pltpu.VMEM((3, N), jnp.float32),
        pltpu.VMEM((3, _RPW), jnp.float32),
        pltpu.VMEM((_CHUNK, C), jnp.float32),
        pltpu.VMEM((_CHUNK, C), jnp.float32),
        pltpu.SemaphoreType.DMA,
        pltpu.SemaphoreType.DMA,
    ],
)
def _sc_gather(feat_hbm, xyz_hbm, idxf_hbm, idxl_hbm, outf_hbm, outx_hbm,
               idxf_v, idxl_v, xyz_v, outx_v, buf0, buf1, sem0, sem1):
    _gather_body(feat_hbm, xyz_hbm, idxf_hbm, idxl_hbm,
                 outf_hbm, outx_hbm,
                 idxf_v, idxl_v, xyz_v, outx_v, buf0, buf1, sem0, sem1)


_IDX_CACHE = None


def _sample_idx() -> np.ndarray:
    global _IDX_CACHE
    if _IDX_CACHE is None:
        cpu = jax.local_devices(backend="cpu")[0]
        with jax.ensure_compile_time_eval(), jax.default_device(cpu):
            keys = jax.random.split(jax.random.key(42), B)
            perm = jax.vmap(lambda k: jax.random.permutation(k, N))(keys)
            _IDX_CACHE = np.asarray(perm[:, :NSAMPLE]).astype(np.int32)
    return _IDX_CACHE


def kernel(xyz, feature):
    idx = _sample_idx()

    idx_flat = idx + (np.arange(B, dtype=np.int32) * N)[:, None]
    idxf = jnp.asarray(idx_flat.reshape(_NW, _NCHUNK, _CHUNK))
    idxl = jnp.asarray(idx.reshape(_NW, _RPW))

    featf = feature.reshape(B * N, C)
    xyzt = jnp.transpose(xyz, (2, 0, 1))

    outf, outx = _sc_gather(featf, xyzt, idxf, idxl)

    feat_s = outf.reshape(B, NSAMPLE, C)
    xyz_s = (outx.reshape(B, 2, 3, _RPW)
                 .transpose(0, 1, 3, 2)
                 .reshape(B, NSAMPLE, 3))
    return (xyz_s, feat_s)


# speedup vs baseline: 5.9515x; 1.1441x over previous
import functools

import jax
import jax.numpy as jnp
import numpy as np
from jax import lax
from jax.experimental import pallas as pl
from jax.experimental.pallas import tpu as pltpu
from jax.experimental.pallas import tpu_sc as plsc

B = 16
N = 16384
C = 256
NSAMPLE = 4096

_NC = 2
_NS = 16
_NW = _NC * _NS
_RPW = B * NSAMPLE // _NW
_CHUNK = 128
_NCHUNK = _RPW // _CHUNK


def _gather_body(feat_hbm, xyz_hbm, idxf_hbm, idxl_hbm,
                 outf_hbm, outx_hbm,
                 idxf_v, idxl_v, outx_v,
                 gsem0, gsem1, gsem2, ssem0, ssem1, ssem2):
    wid = lax.axis_index("s") * _NC + lax.axis_index("c")
    b = wid // 2

    pltpu.sync_copy(idxf_hbm.at[wid], idxf_v)
    pltpu.sync_copy(idxl_hbm.at[wid], idxl_v)

    gsems = (gsem0, gsem1, gsem2)
    ssems = (ssem0, ssem1, ssem2)
    row_base = wid * _RPW

    def _feature_phase(buf0, buf1, buf2):
        bufs = (buf0, buf1, buf2)
        cps = [None, None, None]
        scs = [None, None, None]
        for k in range(2):
            cps[k] = pltpu.async_copy(
                feat_hbm.at[idxf_v.at[k]], bufs[k], gsems[k])
        for i in range(_NCHUNK):
            cps[i % 3].wait()
            scs[i % 3] = pltpu.async_copy(
                bufs[i % 3],
                outf_hbm.at[pl.ds(row_base + i * _CHUNK, _CHUNK)],
                ssems[i % 3])
            nxt = i + 2
            if nxt < _NCHUNK:
                if i >= 1:
                    scs[(i - 1) % 3].wait()
                cps[nxt % 3] = pltpu.async_copy(
                    feat_hbm.at[idxf_v.at[nxt]], bufs[nxt % 3], gsems[nxt % 3])
        for i in range(_NCHUNK - 3, _NCHUNK):
            scs[i % 3].wait()

    pl.run_scoped(_feature_phase,
                  pltpu.VMEM((_CHUNK, C), jnp.float32),
                  pltpu.VMEM((_CHUNK, C), jnp.float32),
                  pltpu.VMEM((_CHUNK, C), jnp.float32))

    def _xyz_phase(xyz_v):
        for d in range(3):
            pltpu.sync_copy(xyz_hbm.at[d, pl.ds(b, 1)],
                            xyz_v.at[pl.ds(d, 1)])

        def _xyz_body(j, carry):
            ids = idxl_v[pl.ds(j * 16, 16)]
            for d in range(3):
                dvec = jnp.full((16,), d, jnp.int32)
                outx_v[d, pl.ds(j * 16, 16)] = plsc.load_gather(
                    xyz_v, [dvec, ids])
            return carry

        lax.fori_loop(0, _RPW // 16, _xyz_body, 0)
        pltpu.sync_copy(outx_v, outx_hbm.at[wid])

    pl.run_scoped(_xyz_phase, pltpu.VMEM((3, N), jnp.float32))


@functools.partial(
    pl.kernel,
    out_type=(
        jax.ShapeDtypeStruct((B * NSAMPLE, C), jnp.float32),
        jax.ShapeDtypeStruct((_NW, 3, _RPW), jnp.float32),
    ),
    mesh=plsc.VectorSubcoreMesh(core_axis_name="c", subcore_axis_name="s"),
    compiler_params=pltpu.CompilerParams(needs_layout_passes=False),
    scratch_types=[
        pltpu.VMEM((_NCHUNK, _CHUNK), jnp.int32),
        pltpu.VMEM((_RPW,), jnp.int32),
        pltpu.VMEM((3, _RPW), jnp.float32),
        pltpu.SemaphoreType.DMA,
        pltpu.SemaphoreType.DMA,
        pltpu.SemaphoreType.DMA,
        pltpu.SemaphoreType.DMA,
        pltpu.SemaphoreType.DMA,
        pltpu.SemaphoreType.DMA,
    ],
)
def _sc_gather(feat_hbm, xyz_hbm, idxf_hbm, idxl_hbm, outf_hbm, outx_hbm,
               idxf_v, idxl_v, outx_v,
               gsem0, gsem1, gsem2, ssem0, ssem1, ssem2):
    _gather_body(feat_hbm, xyz_hbm, idxf_hbm, idxl_hbm,
                 outf_hbm, outx_hbm,
                 idxf_v, idxl_v, outx_v,
                 gsem0, gsem1, gsem2, ssem0, ssem1, ssem2)


_IDX_CACHE = None


def _sample_idx() -> np.ndarray:
    global _IDX_CACHE
    if _IDX_CACHE is None:
        cpu = jax.local_devices(backend="cpu")[0]
        with jax.ensure_compile_time_eval(), jax.default_device(cpu):
            keys = jax.random.split(jax.random.key(42), B)
            perm = jax.vmap(lambda k: jax.random.permutation(k, N))(keys)
            _IDX_CACHE = np.asarray(perm[:, :NSAMPLE]).astype(np.int32)
    return _IDX_CACHE


def kernel(xyz, feature):
    idx = _sample_idx()

    idx_flat = idx + (np.arange(B, dtype=np.int32) * N)[:, None]
    idxf = jnp.asarray(idx_flat.reshape(_NW, _NCHUNK, _CHUNK))
    idxl = jnp.asarray(idx.reshape(_NW, _RPW))

    featf = feature.reshape(B * N, C)
    xyzt = jnp.transpose(xyz, (2, 0, 1))

    outf, outx = _sc_gather(featf, xyzt, idxf, idxl)

    feat_s = outf.reshape(B, NSAMPLE, C)
    xyz_s = (outx.reshape(B, 2, 3, _RPW)
                 .transpose(0, 1, 3, 2)
                 .reshape(B, NSAMPLE, 3))
    return (xyz_s, feat_s)


# speedup vs baseline: 6.2505x; 1.0503x over previous
import functools

import jax
import jax.numpy as jnp
import numpy as np
from jax import lax
from jax.experimental import pallas as pl
from jax.experimental.pallas import tpu as pltpu
from jax.experimental.pallas import tpu_sc as plsc

B = 16
N = 16384
C = 256
NSAMPLE = 4096

_NC = 2
_NS = 16
_NW = _NC * _NS
_RPW = B * NSAMPLE // _NW
_CHUNK = 128
_NCHUNK = _RPW // _CHUNK


def _gather_body(feat_hbm, xyz_hbm, idxf_hbm,
                 outf_hbm, outx_hbm,
                 idxf_v, outx_v,
                 gsem0, gsem1, gsem2, ssem0, ssem1, ssem2):
    wid = lax.axis_index("s") * _NC + lax.axis_index("c")
    b = wid // 2
    h = wid % 2

    pltpu.sync_copy(idxf_hbm.at[wid], idxf_v)

    gsems = (gsem0, gsem1, gsem2)
    ssems = (ssem0, ssem1, ssem2)
    row_base = wid * _RPW

    def _feature_phase(buf0, buf1, buf2):
        bufs = (buf0, buf1, buf2)
        cps = [None, None, None]
        scs = [None, None, None]
        for k in range(2):
            cps[k] = pltpu.async_copy(
                feat_hbm.at[idxf_v.at[k]], bufs[k], gsems[k])
        for i in range(_NCHUNK):
            cps[i % 3].wait()
            scs[i % 3] = pltpu.async_copy(
                bufs[i % 3],
                outf_hbm.at[pl.ds(row_base + i * _CHUNK, _CHUNK)],
                ssems[i % 3])
            nxt = i + 2
            if nxt < _NCHUNK:
                if i >= 1:
                    scs[(i - 1) % 3].wait()
                cps[nxt % 3] = pltpu.async_copy(
                    feat_hbm.at[idxf_v.at[nxt]], bufs[nxt % 3], gsems[nxt % 3])
        for i in range(_NCHUNK - 3, _NCHUNK):
            scs[i % 3].wait()

    pl.run_scoped(_feature_phase,
                  pltpu.VMEM((_CHUNK, C), jnp.float32),
                  pltpu.VMEM((_CHUNK, C), jnp.float32),
                  pltpu.VMEM((_CHUNK, C), jnp.float32))

    def _xyz_phase(xyz_v):
        for d in range(3):
            pltpu.sync_copy(xyz_hbm.at[d, pl.ds(b, 1)],
                            xyz_v.at[pl.ds(d, 1)])

        def _xyz_body(j, carry):
            r = j >> 3
            c = (j & 7) * 16
            ids = idxf_v[r, pl.ds(c, 16)] - b * N
            for d in range(3):
                dvec = jnp.full((16,), d, jnp.int32)
                outx_v[d, pl.ds(j * 16, 16)] = plsc.load_gather(
                    xyz_v, [dvec, ids])
            return carry

        lax.fori_loop(0, _RPW // 16, _xyz_body, 0)
        for d in range(3):
            pltpu.sync_copy(outx_v.at[pl.ds(d, 1)],
                            outx_hbm.at[d, pl.ds(b, 1),
                                        pl.ds(h * _RPW, _RPW)])

    pl.run_scoped(_xyz_phase, pltpu.VMEM((3, N), jnp.float32))


@functools.partial(
    pl.kernel,
    out_type=(
        jax.ShapeDtypeStruct((B * NSAMPLE, C), jnp.float32),
        jax.ShapeDtypeStruct((3, B, NSAMPLE), jnp.float32),
    ),
    mesh=plsc.VectorSubcoreMesh(core_axis_name="c", subcore_axis_name="s"),
    compiler_params=pltpu.CompilerParams(needs_layout_passes=False),
    scratch_types=[
        pltpu.VMEM((_NCHUNK, _CHUNK), jnp.int32),
        pltpu.VMEM((3, _RPW), jnp.float32),
        pltpu.SemaphoreType.DMA,
        pltpu.SemaphoreType.DMA,
        pltpu.SemaphoreType.DMA,
        pltpu.SemaphoreType.DMA,
        pltpu.SemaphoreType.DMA,
        pltpu.SemaphoreType.DMA,
    ],
)
def _sc_gather(feat_hbm, xyz_hbm, idxf_hbm, outf_hbm, outx_hbm,
               idxf_v, outx_v,
               gsem0, gsem1, gsem2, ssem0, ssem1, ssem2):
    _gather_body(feat_hbm, xyz_hbm, idxf_hbm,
                 outf_hbm, outx_hbm,
                 idxf_v, outx_v,
                 gsem0, gsem1, gsem2, ssem0, ssem1, ssem2)


_IDX_CACHE = None


def _sample_idx() -> np.ndarray:
    global _IDX_CACHE
    if _IDX_CACHE is None:
        cpu = jax.local_devices(backend="cpu")[0]
        with jax.ensure_compile_time_eval(), jax.default_device(cpu):
            keys = jax.random.split(jax.random.key(42), B)
            perm = jax.vmap(lambda k: jax.random.permutation(k, N))(keys)
            _IDX_CACHE = np.asarray(perm[:, :NSAMPLE]).astype(np.int32)
    return _IDX_CACHE


def kernel(xyz, feature):
    idx = _sample_idx()

    idx_flat = idx + (np.arange(B, dtype=np.int32) * N)[:, None]
    idxf = jnp.asarray(idx_flat.reshape(_NW, _NCHUNK, _CHUNK))

    featf = feature.reshape(B * N, C)
    xyzt = jnp.transpose(xyz, (2, 0, 1))

    outf, outx = _sc_gather(featf, xyzt, idxf)

    feat_s = outf.reshape(B, NSAMPLE, C)
    xyz_s = jnp.transpose(outx, (1, 2, 0))
    return (xyz_s, feat_s)


# speedup vs baseline: 6.3092x; 1.0094x over previous
import functools

import jax
import jax.numpy as jnp
import numpy as np
from jax import lax
from jax.experimental import pallas as pl
from jax.experimental.pallas import tpu as pltpu
from jax.experimental.pallas import tpu_sc as plsc

B = 16
N = 16384
C = 256
NSAMPLE = 4096

_NC = 2
_NS = 16
_NW = _NC * _NS
_RPW = B * NSAMPLE // _NW
_CHUNK = 64
_NCHUNK = _RPW // _CHUNK


def _gather_body(feat_hbm, xyz_hbm, idxf_hbm,
                 outf_hbm, outx_hbm,
                 idxf_v, outx_v, xyz_v, buf0, buf1, buf2,
                 gsem0, gsem1, gsem2, ssem0, ssem1, ssem2):
    wid = lax.axis_index("s") * _NC + lax.axis_index("c")
    b = wid // 2
    h = wid % 2

    pltpu.sync_copy(idxf_hbm.at[wid], idxf_v)

    gsems = (gsem0, gsem1, gsem2)
    ssems = (ssem0, ssem1, ssem2)
    row_base = wid * _RPW
    bufs = (buf0, buf1, buf2)
    cps = [None, None, None]
    scs = [None, None, None]

    for k in range(2):
        cps[k] = pltpu.async_copy(
            feat_hbm.at[idxf_v.at[k]], bufs[k], gsems[k])

    for d in range(3):
        pltpu.sync_copy(xyz_hbm.at[d, pl.ds(b, 1)],
                        xyz_v.at[pl.ds(d, 1)])

    def _xyz_body(j, carry):
        r = j >> 2
        c = (j & 3) * 16
        ids = idxf_v[r, pl.ds(c, 16)] - b * N
        for d in range(3):
            dvec = jnp.full((16,), d, jnp.int32)
            outx_v[d, pl.ds(j * 16, 16)] = plsc.load_gather(
                xyz_v, [dvec, ids])
        return carry

    lax.fori_loop(0, _RPW // 16, _xyz_body, 0)
    for d in range(3):
        pltpu.sync_copy(outx_v.at[pl.ds(d, 1)],
                        outx_hbm.at[d, pl.ds(b, 1),
                                    pl.ds(h * _RPW, _RPW)])

    for i in range(_NCHUNK):
        cps[i % 3].wait()
        scs[i % 3] = pltpu.async_copy(
            bufs[i % 3],
            outf_hbm.at[pl.ds(row_base + i * _CHUNK, _CHUNK)],
            ssems[i % 3])
        nxt = i + 2
        if nxt < _NCHUNK:
            if i >= 1:
                scs[(i - 1) % 3].wait()
            cps[nxt % 3] = pltpu.async_copy(
                feat_hbm.at[idxf_v.at[nxt]], bufs[nxt % 3], gsems[nxt % 3])
    for i in range(_NCHUNK - 3, _NCHUNK):
        scs[i % 3].wait()


@functools.partial(
    pl.kernel,
    out_type=(
        jax.ShapeDtypeStruct((B * NSAMPLE, C), jnp.float32),
        jax.ShapeDtypeStruct((3, B, NSAMPLE), jnp.float32),
    ),
    mesh=plsc.VectorSubcoreMesh(core_axis_name="c", subcore_axis_name="s"),
    compiler_params=pltpu.CompilerParams(needs_layout_passes=False),
    scratch_types=[
        pltpu.VMEM((_NCHUNK, _CHUNK), jnp.int32),
        pltpu.VMEM((3, _RPW), jnp.float32),
        pltpu.VMEM((3, N), jnp.float32),
        pltpu.VMEM((_CHUNK, C), jnp.float32),
        pltpu.VMEM((_CHUNK, C), jnp.float32),
        pltpu.VMEM((_CHUNK, C), jnp.float32),
        pltpu.SemaphoreType.DMA,
        pltpu.SemaphoreType.DMA,
        pltpu.SemaphoreType.DMA,
        pltpu.SemaphoreType.DMA,
        pltpu.SemaphoreType.DMA,
        pltpu.SemaphoreType.DMA,
    ],
)
def _sc_gather(feat_hbm, xyz_hbm, idxf_hbm, outf_hbm, outx_hbm,
               idxf_v, outx_v, xyz_v, buf0, buf1, buf2,
               gsem0, gsem1, gsem2, ssem0, ssem1, ssem2):
    _gather_body(feat_hbm, xyz_hbm, idxf_hbm,
                 outf_hbm, outx_hbm,
                 idxf_v, outx_v, xyz_v, buf0, buf1, buf2,
                 gsem0, gsem1, gsem2, ssem0, ssem1, ssem2)


_IDX_CACHE = None


def _sample_idx() -> np.ndarray:
    global _IDX_CACHE
    if _IDX_CACHE is None:
        cpu = jax.local_devices(backend="cpu")[0]
        with jax.ensure_compile_time_eval(), jax.default_device(cpu):
            keys = jax.random.split(jax.random.key(42), B)
            perm = jax.vmap(lambda k: jax.random.permutation(k, N))(keys)
            _IDX_CACHE = np.asarray(perm[:, :NSAMPLE]).astype(np.int32)
    return _IDX_CACHE


def kernel(xyz, feature):
    idx = _sample_idx()

    idx_flat = idx + (np.arange(B, dtype=np.int32) * N)[:, None]
    idxf = jnp.asarray(idx_flat.reshape(_NW, _NCHUNK, _CHUNK))

    featf = feature.reshape(B * N, C)
    xyzt = jnp.transpose(xyz, (2, 0, 1))

    outf, outx = _sc_gather(featf, xyzt, idxf)

    feat_s = outf.reshape(B, NSAMPLE, C)
    xyz_s = jnp.transpose(outx, (1, 2, 0))
    return (xyz_s, feat_s)
